# baseline (speedup 1.0000x reference)
import functools

import numpy as np
import jax
import jax.numpy as jnp
from jax import lax
from jax.experimental import pallas as pl
from jax.experimental.pallas import tpu as pltpu
from jax.experimental.pallas import tpu_sc as plsc

RADIUS = 0.4
K = 64
B, N, S, C = 4, 8192, 1024, 128
NW = N // 16
NG = NW // 16
CH = C + 3
ROWS = B * S
PAIRS = ROWS // 2
S2 = S // 2
XROWS = B * 3 * S2
S_BLK = 128
R2 = RADIUS * RADIUS
TS = 16
TSK = TS * K


def _seg_matrix() -> np.ndarray:
    i = np.arange(N)
    w = np.arange(NW)
    segf = np.where((i[:, None] // 16) == w[None, :],
                    (2.0 ** (i % 16))[:, None], 0.0).astype(np.float32)
    return segf.astype(jnp.bfloat16)


_SEG = _seg_matrix()


def _pack_body(nx_ref, xyzT_ref, seg_ref, out_ref):
    nx = nx_ref[0]
    p = xyzT_ref[0]
    acc = None
    for c in range(3):
        qc = nx[:, c:c + 1]
        pc = p[c:c + 1, :]
        d = qc - pc
        acc = d * d if acc is None else acc + d * d
    m = (acc < R2).astype(jnp.bfloat16)
    w = lax.dot_general(m, seg_ref[...], (((1,), (0,)), ((), ())),
                        preferred_element_type=jnp.float32)
    out_ref[...] = w.astype(jnp.int32)


def _pack_words(new_xyz, xyzT, seg):
    return pl.pallas_call(
        _pack_body,
        grid=(B, S // S_BLK),
        in_specs=[
            pl.BlockSpec((1, S_BLK, 3), lambda b, sb: (b, sb, 0)),
            pl.BlockSpec((1, 3, N), lambda b, sb: (b, 0, 0)),
            pl.BlockSpec((N, NW), lambda b, sb: (0, 0)),
        ],
        out_specs=pl.BlockSpec((S_BLK, NW),
                               lambda b, sb: (b * (S // S_BLK) + sb, 0)),
        out_shape=jax.ShapeDtypeStruct((ROWS, NW), jnp.int32),
    )(new_xyz, xyzT, seg)


@functools.cache
def _sc_group(b0, nb):
    mesh = plsc.VectorSubcoreMesh(core_axis_name="c", subcore_axis_name="s")
    nwork = mesh.num_cores * mesh.num_subcores
    ppw = (nb * S2) // nwork
    qpw = ppw * 2
    HR = nb * S * K
    XR = nb * 3 * S2

    def body(words_hbm, nxp_hbm, xyzp_hbm, feat_hbm, stag_hbm,
             wv, idxv, lidxv0, lidxv1, gidxv0, gidxv1, xv, gv0, gv1,
             xyzv0, xyzv1, nxav, xiv0, xiv1,
             semg0, semg1, sems):
        iota16 = lax.iota(jnp.int32, 16)
        wid = lax.axis_index("s") * mesh.num_cores + lax.axis_index("c")
        batch = b0 + (wid * qpw) // S
        bn = batch * N
        pltpu.sync_copy(xyzp_hbm.at[pl.ds(batch * N * 4, N * 4)], xv)
        pltpu.sync_copy(
            nxp_hbm.at[pl.ds((b0 * S + wid * qpw) * 16, qpw * 16)], nxav)

        def select(q, lidxv):
            idxv[q, pl.ds(0, 16)] = jnp.zeros((16,), jnp.int32)

            def sel_cond(carry):
                g, ws, cnt = carry
                return (g < NG) & (cnt < K)

            def sel_body(carry):
                g, ws, cnt = carry
                nz = ws != 0
                nzp = plsc.all_reduce_population_count(nz)[0]
                wlv = plsc.all_reduce_ffs(nz)
                wl_safe = jnp.minimum(wlv, 15)
                wb = plsc.load_gather(wv, [jnp.full((16,), q, jnp.int32),
                                           16 * g + wl_safe])
                bits = (wb >> iota16) & 1
                msk = (bits == 1) & (nzp > 0)
                vals = (g * 16 + wlv) * 16 + iota16
                plsc.store_compressed(idxv.at[q].at[pl.ds(cnt, 16)],
                                      vals, mask=msk)
                cnt = cnt + plsc.all_reduce_population_count(msk)[0]
                ws = jnp.where(iota16 == wlv, 0, ws)
                adv = plsc.all_reduce_population_count(ws != 0)[0] == 0
                gn = g + jnp.where(adv, jnp.int32(1), jnp.int32(0))
                wsl = wv[q, pl.ds(16 * jnp.minimum(gn, NG - 1), 16)]
                ws = jnp.where(adv, wsl, ws)
                return gn, ws, cnt

            ws0 = wv[q, pl.ds(0, 16)]
            _, _, cnt = lax.while_loop(
                sel_cond, sel_body, (jnp.int32(0), ws0, jnp.int32(0)))

            idx0 = idxv[q, pl.ds(0, 16)]
            first = idx0[0]
            qh = q % 2
            for j in range(K // 16):
                v = idxv[q, pl.ds(16 * j, 16)]
                pos = 16 * j + iota16
                lidxv[pl.ds(qh * K + 16 * j, 16)] = jnp.where(
                    pos < cnt, v, first)

        def xyz_chan(u, pair_off, lidxv, xyzv, xiv):
            pair = wid * ppw + 2 * u + pair_off
            q0 = (2 * u + pair_off) * 2
            xbase = HR + (batch - b0) * 3 * S2 + pair % S2
            xiv[...] = jnp.where(iota16 < 3, xbase + iota16 * S2,
                                 HR + XR + (iota16 - 3))
            nx0 = nxav[pl.ds(q0 * 16, 16)]
            nx1 = nxav[pl.ds(q0 * 16 + 16, 16)]
            for c in range(3):
                for j in range(8):
                    sub = nx0[c] if j < 4 else nx1[c]
                    rows = lidxv[pl.ds(16 * j, 16)]
                    vals = plsc.load_gather(xv, [rows * 4 + c])
                    xyzv[c, pl.ds(16 * j, 16)] = vals * (1.0 / RADIUS) - sub

        def u_body(u, _):
            pair0 = wid * ppw + 2 * u
            pltpu.sync_copy(
                words_hbm.at[pl.ds((b0 * S2 + pair0) * 2, 4)], wv)

            select(0, lidxv0)
            select(1, lidxv0)
            for j in range(8):
                gidxv0[pl.ds(16 * j, 16)] = lidxv0[pl.ds(16 * j, 16)] + bn
            hg0 = pltpu.async_copy(feat_hbm.at[gidxv0], gv0, semg0)

            select(2, lidxv1)
            select(3, lidxv1)
            for j in range(8):
                gidxv1[pl.ds(16 * j, 16)] = lidxv1[pl.ds(16 * j, 16)] + bn
            hg1 = pltpu.async_copy(feat_hbm.at[gidxv1], gv1, semg1)

            hg0.wait()
            xyz_chan(u, 0, lidxv0, xyzv0, xiv0)
            h0a = pltpu.async_copy(gv0, stag_hbm.at[pl.ds(pair0 * 128, 128)],
                                   sems)
            h0b = pltpu.async_copy(xyzv0, stag_hbm.at[xiv0], sems)

            hg1.wait()
            xyz_chan(u, 1, lidxv1, xyzv1, xiv1)
            h1a = pltpu.async_copy(gv1,
                                   stag_hbm.at[pl.ds((pair0 + 1) * 128, 128)],
                                   sems)
            h1b = pltpu.async_copy(xyzv1, stag_hbm.at[xiv1], sems)

            h0a.wait()
            h0b.wait()
            h1a.wait()
            h1b.wait()
            return 0

        lax.fori_loop(0, ppw // 2, u_body, 0)

    return pl.kernel(
        body,
        out_type=jax.ShapeDtypeStruct((HR + XR + 16, C), jnp.float32),
        mesh=mesh,
        compiler_params=pltpu.CompilerParams(needs_layout_passes=False),
        scratch_types=[
            pltpu.VMEM((4, NW), jnp.int32),
            pltpu.VMEM((4, 96), jnp.int32),
            pltpu.VMEM((2 * K,), jnp.int32),
            pltpu.VMEM((2 * K,), jnp.int32),
            pltpu.VMEM((2 * K,), jnp.int32),
            pltpu.VMEM((2 * K,), jnp.int32),
            pltpu.VMEM((N * 4,), jnp.float32),
            pltpu.VMEM((2 * K, C), jnp.float32),
            pltpu.VMEM((2 * K, C), jnp.float32),
            pltpu.VMEM((16, 2 * K), jnp.float32),
            pltpu.VMEM((16, 2 * K), jnp.float32),
            pltpu.VMEM((ppw * 32,), jnp.float32),
            pltpu.VMEM((16,), jnp.int32),
            pltpu.VMEM((16,), jnp.int32),
            pltpu.SemaphoreType.DMA,
            pltpu.SemaphoreType.DMA,
            pltpu.SemaphoreType.DMA,
        ],
    )


def _asm_body(stag_ref, xyzc_ref, out_ref):
    x = stag_ref[0]
    out_ref[0, 0:3, :] = xyzc_ref[0]
    out_ref[0, 3:CH, :] = jnp.swapaxes(x, 0, 1)


def _assemble(stag, xyzc):
    stag3 = stag.reshape(B, S * K, C)
    xyz3 = xyzc.reshape(B, 3, S * K)
    return pl.pallas_call(
        _asm_body,
        grid=(B, (S * K) // TSK),
        in_specs=[
            pl.BlockSpec((1, TSK, C), lambda b, t: (b, t, 0)),
            pl.BlockSpec((1, 3, TSK), lambda b, t: (b, 0, t)),
        ],
        out_specs=pl.BlockSpec((1, CH, TSK), lambda b, t: (b, 0, t)),
        out_shape=jax.ShapeDtypeStruct((B, CH, S * K), jnp.float32),
    )(stag3, xyz3)


@jax.jit
def _pipeline(xyz, new_xyz, features):
    xyzT = jnp.transpose(xyz, (0, 2, 1))
    words = _pack_words(new_xyz, xyzT, jnp.asarray(_SEG))
    featT = jnp.transpose(features, (0, 2, 1)).reshape(B * N, C)
    xyzp = jnp.concatenate(
        [xyz, jnp.zeros((B, N, 1), jnp.float32)], axis=-1).reshape(-1)
    nxp = jnp.pad(new_xyz.reshape(ROWS, 3) * (1.0 / RADIUS),
                  ((0, 0), (0, 13))).reshape(-1)
    nb = 2
    halves = []
    for b0 in range(0, B, nb):
        h = _sc_group(b0, nb)(words, nxp, xyzp, featT)
        hr = nb * S * K
        xr = nb * 3 * S2
        stagf = h[:hr].reshape(nb, S, K, C)
        featc = jnp.transpose(stagf, (0, 3, 1, 2))
        xyzc = h[hr:hr + xr].reshape(nb, 3, S, K)
        halves.append(jnp.concatenate([xyzc, featc], axis=1))
    return jnp.concatenate(halves, axis=0)


def kernel(xyz, new_xyz, features):
    return _pipeline(xyz, new_xyz, features)


# speedup vs baseline: 1.0172x; 1.0172x over previous
import functools

import numpy as np
import jax
import jax.numpy as jnp
from jax import lax
from jax.experimental import pallas as pl
from jax.experimental.pallas import tpu as pltpu
from jax.experimental.pallas import tpu_sc as plsc

RADIUS = 0.4
K = 64
B, N, S, C = 4, 8192, 1024, 128
NW = N // 16
NG = NW // 16
CH = C + 3
ROWS = B * S
PAIRS = ROWS // 2
S2 = S // 2
XROWS = B * 3 * S2
S_BLK = 128
R2 = RADIUS * RADIUS
TS = 16
TSK = TS * K


def _seg_matrix() -> np.ndarray:
    i = np.arange(N)
    w = np.arange(NW)
    segf = np.where((i[:, None] // 16) == w[None, :],
                    (2.0 ** (i % 16))[:, None], 0.0).astype(np.float32)
    return segf.astype(jnp.bfloat16)


_SEG = _seg_matrix()


def _pack_body(nx_ref, xyzT_ref, seg_ref, out_ref):
    nx = nx_ref[0]
    p = xyzT_ref[0]
    acc = None
    for c in range(3):
        qc = nx[:, c:c + 1]
        pc = p[c:c + 1, :]
        d = qc - pc
        acc = d * d if acc is None else acc + d * d
    m = (acc < R2).astype(jnp.bfloat16)
    w = lax.dot_general(m, seg_ref[...], (((1,), (0,)), ((), ())),
                        preferred_element_type=jnp.float32)
    out_ref[...] = w.astype(jnp.int32)


def _pack_words(new_xyz, xyzT, seg):
    return pl.pallas_call(
        _pack_body,
        grid=(B, S // S_BLK),
        in_specs=[
            pl.BlockSpec((1, S_BLK, 3), lambda b, sb: (b, sb, 0)),
            pl.BlockSpec((1, 3, N), lambda b, sb: (b, 0, 0)),
            pl.BlockSpec((N, NW), lambda b, sb: (0, 0)),
        ],
        out_specs=pl.BlockSpec((S_BLK, NW),
                               lambda b, sb: (b * (S // S_BLK) + sb, 0)),
        out_shape=jax.ShapeDtypeStruct((ROWS, NW), jnp.int32),
    )(new_xyz, xyzT, seg)


@functools.cache
def _sc_group():
    mesh = plsc.VectorSubcoreMesh(core_axis_name="c", subcore_axis_name="s")
    nwork = mesh.num_cores * mesh.num_subcores
    ppw = PAIRS // nwork
    qpw = ppw * 2

    def body(words_hbm, nxp_hbm, xyzp_hbm, feat_hbm, stag_hbm,
             wv, idxv, lidxv0, lidxv1, gidxv0, gidxv1, xv, gv0, gv1,
             xyzv0, xyzv1, nxav, xiv0, xiv1,
             semg0, semg1, sems):
        iota16 = lax.iota(jnp.int32, 16)
        wid = lax.axis_index("s") * mesh.num_cores + lax.axis_index("c")
        batch = (wid * qpw) // S
        bn = batch * N
        hx = pltpu.async_copy(xyzp_hbm.at[pl.ds(batch * N * 4, N * 4)], xv,
                              semg0)
        hn = pltpu.async_copy(nxp_hbm.at[pl.ds(wid * qpw * 16, qpw * 16)],
                              nxav, semg1)
        hx.wait()
        hn.wait()

        def select(q, lidxv):
            idxv[q, pl.ds(0, 16)] = jnp.zeros((16,), jnp.int32)

            def sel_cond(carry):
                g, ws, cnt = carry
                return (g < NG) & (cnt < K)

            def sel_body(carry):
                g, ws, cnt = carry
                nz = ws != 0
                nzp = plsc.all_reduce_population_count(nz)[0]
                wlv = plsc.all_reduce_ffs(nz)
                wl_safe = jnp.minimum(wlv, 15)
                wb = plsc.load_gather(wv, [jnp.full((16,), q, jnp.int32),
                                           16 * g + wl_safe])
                bits = (wb >> iota16) & 1
                msk = (bits == 1) & (nzp > 0)
                vals = (g * 16 + wlv) * 16 + iota16
                plsc.store_compressed(idxv.at[q].at[pl.ds(cnt, 16)],
                                      vals, mask=msk)
                cnt = cnt + plsc.all_reduce_population_count(msk)[0]
                ws = jnp.where(iota16 == wlv, 0, ws)
                adv = plsc.all_reduce_population_count(ws != 0)[0] == 0
                gn = g + jnp.where(adv, jnp.int32(1), jnp.int32(0))
                wsl = wv[q, pl.ds(16 * jnp.minimum(gn, NG - 1), 16)]
                ws = jnp.where(adv, wsl, ws)
                return gn, ws, cnt

            ws0 = wv[q, pl.ds(0, 16)]
            _, _, cnt = lax.while_loop(
                sel_cond, sel_body, (jnp.int32(0), ws0, jnp.int32(0)))

            idx0 = idxv[q, pl.ds(0, 16)]
            first = idx0[0]
            qh = q % 2
            for j in range(K // 16):
                v = idxv[q, pl.ds(16 * j, 16)]
                pos = 16 * j + iota16
                lidxv[pl.ds(qh * K + 16 * j, 16)] = jnp.where(
                    pos < cnt, v, first)

        def xyz_chan(u, pair_off, lidxv, xyzv, xiv):
            pair = wid * ppw + 2 * u + pair_off
            q0 = (2 * u + pair_off) * 2
            xbase = ROWS * K + batch * 3 * S2 + pair % S2
            xiv[...] = jnp.where(iota16 < 3, xbase + iota16 * S2,
                                 ROWS * K + XROWS + (iota16 - 3))
            nx0 = nxav[pl.ds(q0 * 16, 16)]
            nx1 = nxav[pl.ds(q0 * 16 + 16, 16)]
            for c in range(3):
                for j in range(8):
                    sub = nx0[c] if j < 4 else nx1[c]
                    rows = lidxv[pl.ds(16 * j, 16)]
                    vals = plsc.load_gather(xv, [rows * 4 + c])
                    xyzv[c, pl.ds(16 * j, 16)] = vals * (1.0 / RADIUS) - sub

        def u_body(u, _):
            pair0 = wid * ppw + 2 * u
            pltpu.sync_copy(words_hbm.at[pl.ds(pair0 * 2, 4)], wv)

            select(0, lidxv0)
            select(1, lidxv0)
            for j in range(8):
                gidxv0[pl.ds(16 * j, 16)] = lidxv0[pl.ds(16 * j, 16)] + bn
            hg0 = pltpu.async_copy(feat_hbm.at[gidxv0], gv0, semg0)

            select(2, lidxv1)
            select(3, lidxv1)
            for j in range(8):
                gidxv1[pl.ds(16 * j, 16)] = lidxv1[pl.ds(16 * j, 16)] + bn
            hg1 = pltpu.async_copy(feat_hbm.at[gidxv1], gv1, semg1)

            hg0.wait()
            xyz_chan(u, 0, lidxv0, xyzv0, xiv0)
            h0a = pltpu.async_copy(gv0, stag_hbm.at[pl.ds(pair0 * 128, 128)],
                                   sems)
            h0b = pltpu.async_copy(xyzv0, stag_hbm.at[xiv0], sems)

            hg1.wait()
            xyz_chan(u, 1, lidxv1, xyzv1, xiv1)
            h1a = pltpu.async_copy(gv1,
                                   stag_hbm.at[pl.ds((pair0 + 1) * 128, 128)],
                                   sems)
            h1b = pltpu.async_copy(xyzv1, stag_hbm.at[xiv1], sems)

            h0a.wait()
            h0b.wait()
            h1a.wait()
            h1b.wait()
            return 0

        lax.fori_loop(0, ppw // 2, u_body, 0)

    return pl.kernel(
        body,
        out_type=jax.ShapeDtypeStruct((ROWS * K + XROWS + 16, C),
                                      jnp.float32),
        mesh=mesh,
        compiler_params=pltpu.CompilerParams(needs_layout_passes=False),
        scratch_types=[
            pltpu.VMEM((4, NW), jnp.int32),
            pltpu.VMEM((4, 96), jnp.int32),
            pltpu.VMEM((2 * K,), jnp.int32),
            pltpu.VMEM((2 * K,), jnp.int32),
            pltpu.VMEM((2 * K,), jnp.int32),
            pltpu.VMEM((2 * K,), jnp.int32),
            pltpu.VMEM((N * 4,), jnp.float32),
            pltpu.VMEM((2 * K, C), jnp.float32),
            pltpu.VMEM((2 * K, C), jnp.float32),
            pltpu.VMEM((16, 2 * K), jnp.float32),
            pltpu.VMEM((16, 2 * K), jnp.float32),
            pltpu.VMEM((PAIRS // nwork * 32,), jnp.float32),
            pltpu.VMEM((16,), jnp.int32),
            pltpu.VMEM((16,), jnp.int32),
            pltpu.SemaphoreType.DMA,
            pltpu.SemaphoreType.DMA,
            pltpu.SemaphoreType.DMA,
        ],
    )


def _asm_body(stag_ref, xyzc_ref, out_ref):
    x = stag_ref[0]
    out_ref[0, 0:3, :] = xyzc_ref[0]
    out_ref[0, 3:CH, :] = jnp.swapaxes(x, 0, 1)


def _assemble(stag, xyzc):
    stag3 = stag.reshape(B, S * K, C)
    xyz3 = xyzc.reshape(B, 3, S * K)
    return pl.pallas_call(
        _asm_body,
        grid=(B, (S * K) // TSK),
        in_specs=[
            pl.BlockSpec((1, TSK, C), lambda b, t: (b, t, 0)),
            pl.BlockSpec((1, 3, TSK), lambda b, t: (b, 0, t)),
        ],
        out_specs=pl.BlockSpec((1, CH, TSK), lambda b, t: (b, 0, t)),
        out_shape=jax.ShapeDtypeStruct((B, CH, S * K), jnp.float32),
    )(stag3, xyz3)


@jax.jit
def _pipeline(xyz, new_xyz, features):
    xyzT = jnp.transpose(xyz, (0, 2, 1))
    words = _pack_words(new_xyz, xyzT, jnp.asarray(_SEG))
    featT = jnp.transpose(features, (0, 2, 1)).reshape(B * N, C)
    xyzp = jnp.concatenate(
        [xyz, jnp.zeros((B, N, 1), jnp.float32)], axis=-1).reshape(-1)
    nxp = jnp.pad(new_xyz.reshape(ROWS, 3) * (1.0 / RADIUS),
                  ((0, 0), (0, 13))).reshape(-1)
    merged = _sc_group()(words, nxp, xyzp, featT)
    stagf = merged[:ROWS * K].reshape(B, S, K, C)
    featc = jnp.transpose(stagf, (0, 3, 1, 2))
    xyzc = merged[ROWS * K:ROWS * K + XROWS].reshape(B, 3, S, K)
    return jnp.concatenate([xyzc, featc], axis=1)


def kernel(xyz, new_xyz, features):
    return _pipeline(xyz, new_xyz, features)


# speedup vs baseline: 1.1180x; 1.0991x over previous
import functools

import numpy as np
import jax
import jax.numpy as jnp
from jax import lax
from jax.experimental import pallas as pl
from jax.experimental.pallas import tpu as pltpu
from jax.experimental.pallas import tpu_sc as plsc

RADIUS = 0.4
K = 64
B, N, S, C = 4, 8192, 1024, 128
NW = N // 16
NG = NW // 16
CH = C + 3
ROWS = B * S
PAIRS = ROWS // 2
S2 = S // 2
XROWS = B * 3 * S2
S_BLK = 128
R2 = RADIUS * RADIUS
TS = 16
TSK = TS * K


def _seg_matrix() -> np.ndarray:
    i = np.arange(N)
    w = np.arange(NW)
    segf = np.where((i[:, None] // 16) == w[None, :],
                    (2.0 ** (i % 16))[:, None], 0.0).astype(np.float32)
    return segf.astype(jnp.bfloat16)


_SEG = _seg_matrix()


def _pack_body(nx_ref, xyzT_ref, seg_ref, out_ref):
    nx = nx_ref[0]
    p = xyzT_ref[0]
    acc = None
    for c in range(3):
        qc = nx[:, c:c + 1]
        pc = p[c:c + 1, :]
        d = qc - pc
        acc = d * d if acc is None else acc + d * d
    m = (acc < R2).astype(jnp.bfloat16)
    w = lax.dot_general(m, seg_ref[...], (((1,), (0,)), ((), ())),
                        preferred_element_type=jnp.float32)
    out_ref[...] = w.astype(jnp.int32)


def _pack_words(new_xyz, xyzT, seg):
    return pl.pallas_call(
        _pack_body,
        grid=(B, S // S_BLK),
        in_specs=[
            pl.BlockSpec((1, S_BLK, 3), lambda b, sb: (b, sb, 0)),
            pl.BlockSpec((1, 3, N), lambda b, sb: (b, 0, 0)),
            pl.BlockSpec((N, NW), lambda b, sb: (0, 0)),
        ],
        out_specs=pl.BlockSpec((S_BLK, NW),
                               lambda b, sb: (b * (S // S_BLK) + sb, 0)),
        out_shape=jax.ShapeDtypeStruct((ROWS, NW), jnp.int32),
    )(new_xyz, xyzT, seg)


@functools.cache
def _sc_group():
    mesh = plsc.VectorSubcoreMesh(core_axis_name="c", subcore_axis_name="s")
    nwork = mesh.num_cores * mesh.num_subcores
    ppw = PAIRS // nwork
    qpw = ppw * 2

    def body(words_hbm, nxp_hbm, xyzp_hbm, feat_hbm, stag_hbm,
             wv, idxv, lidxv0, lidxv1, lidxv2, lidxv3,
             gidxv0, gidxv1, gidxv2, gidxv3, xv, gv0, gv1, gv2, gv3,
             xyzv0, xyzv1, xyzv2, xyzv3, nxav, xiv0, xiv1, xiv2, xiv3,
             semg0, semg1, semg2, semg3, sems):
        lidxvs = (lidxv0, lidxv1, lidxv2, lidxv3)
        gidxvs = (gidxv0, gidxv1, gidxv2, gidxv3)
        gvs = (gv0, gv1, gv2, gv3)
        xyzvs = (xyzv0, xyzv1, xyzv2, xyzv3)
        xivs = (xiv0, xiv1, xiv2, xiv3)
        semgs = (semg0, semg1, semg2, semg3)
        iota16 = lax.iota(jnp.int32, 16)
        wid = lax.axis_index("s") * mesh.num_cores + lax.axis_index("c")
        batch = (wid * qpw) // S
        bn = batch * N
        hx = pltpu.async_copy(xyzp_hbm.at[pl.ds(batch * N * 4, N * 4)], xv,
                              semg0)
        hn = pltpu.async_copy(nxp_hbm.at[pl.ds(wid * qpw * 16, qpw * 16)],
                              nxav, semg1)
        hx.wait()
        hn.wait()

        def select(q, lidxv):
            idxv[q, pl.ds(0, 16)] = jnp.zeros((16,), jnp.int32)

            def sel_cond(carry):
                g, ws, cnt = carry
                return (g < NG) & (cnt < K)

            def sel_body(carry):
                g, ws, cnt = carry
                nz = ws != 0
                nzp = plsc.all_reduce_population_count(nz)[0]
                wlv = plsc.all_reduce_ffs(nz)
                wl_safe = jnp.minimum(wlv, 15)
                wb = plsc.load_gather(wv, [jnp.full((16,), q, jnp.int32),
                                           16 * g + wl_safe])
                bits = (wb >> iota16) & 1
                msk = (bits == 1) & (nzp > 0)
                vals = (g * 16 + wlv) * 16 + iota16
                plsc.store_compressed(idxv.at[q].at[pl.ds(cnt, 16)],
                                      vals, mask=msk)
                cnt = cnt + plsc.all_reduce_population_count(msk)[0]
                ws = jnp.where(iota16 == wlv, 0, ws)
                adv = plsc.all_reduce_population_count(ws != 0)[0] == 0
                gn = g + jnp.where(adv, jnp.int32(1), jnp.int32(0))
                wsl = wv[q, pl.ds(16 * jnp.minimum(gn, NG - 1), 16)]
                ws = jnp.where(adv, wsl, ws)
                return gn, ws, cnt

            ws0 = wv[q, pl.ds(0, 16)]
            _, _, cnt = lax.while_loop(
                sel_cond, sel_body, (jnp.int32(0), ws0, jnp.int32(0)))

            idx0 = idxv[q, pl.ds(0, 16)]
            first = idx0[0]
            qh = q % 2
            for j in range(K // 16):
                v = idxv[q, pl.ds(16 * j, 16)]
                pos = 16 * j + iota16
                lidxv[pl.ds(qh * K + 16 * j, 16)] = jnp.where(
                    pos < cnt, v, first)

        def xyz_chan(u, pair_off, lidxv, xyzv, xiv):
            pair = wid * ppw + 4 * u + pair_off
            q0 = (4 * u + pair_off) * 2
            xbase = ROWS * K + batch * 3 * S2 + pair % S2
            xiv[...] = jnp.where(iota16 < 3, xbase + iota16 * S2,
                                 ROWS * K + XROWS + (iota16 - 3))
            nx0 = nxav[pl.ds(q0 * 16, 16)]
            nx1 = nxav[pl.ds(q0 * 16 + 16, 16)]
            for c in range(3):
                for j in range(8):
                    sub = nx0[c] if j < 4 else nx1[c]
                    rows = lidxv[pl.ds(16 * j, 16)]
                    vals = plsc.load_gather(xv, [rows * 4 + c])
                    xyzv[c, pl.ds(16 * j, 16)] = vals * (1.0 / RADIUS) - sub

        def u_body(u, _):
            pair0 = wid * ppw + 4 * u
            pltpu.sync_copy(words_hbm.at[pl.ds(pair0 * 2, 8)], wv)

            hgs = []
            for t in range(4):
                select(2 * t, lidxvs[t])
                select(2 * t + 1, lidxvs[t])
                for j in range(8):
                    gidxvs[t][pl.ds(16 * j, 16)] = (
                        lidxvs[t][pl.ds(16 * j, 16)] + bn)
                hgs.append(pltpu.async_copy(feat_hbm.at[gidxvs[t]],
                                            gvs[t], semgs[t]))

            hws = []
            for t in range(4):
                xyz_chan(u, t, lidxvs[t], xyzvs[t], xivs[t])
                hws.append(pltpu.async_copy(xyzvs[t], stag_hbm.at[xivs[t]],
                                            sems))
            for t in range(4):
                hgs[t].wait()
                hws.append(pltpu.async_copy(
                    gvs[t], stag_hbm.at[pl.ds((pair0 + t) * 128, 128)],
                    sems))
            for h in hws:
                h.wait()
            return 0

        lax.fori_loop(0, ppw // 4, u_body, 0)

    return pl.kernel(
        body,
        out_type=jax.ShapeDtypeStruct((ROWS * K + XROWS + 16, C),
                                      jnp.float32),
        mesh=mesh,
        compiler_params=pltpu.CompilerParams(needs_layout_passes=False),
        scratch_types=(
            [pltpu.VMEM((8, NW), jnp.int32),
             pltpu.VMEM((8, 96), jnp.int32)]
            + [pltpu.VMEM((2 * K,), jnp.int32)] * 8
            + [pltpu.VMEM((N * 4,), jnp.float32)]
            + [pltpu.VMEM((2 * K, C), jnp.float32)] * 4
            + [pltpu.VMEM((16, 2 * K), jnp.float32)] * 4
            + [pltpu.VMEM((PAIRS // nwork * 32,), jnp.float32)]
            + [pltpu.VMEM((16,), jnp.int32)] * 4
            + [pltpu.SemaphoreType.DMA] * 5
        ),
    )


def _asm_body(stag_ref, xyzc_ref, out_ref):
    x = stag_ref[0]
    out_ref[0, 0:3, :] = xyzc_ref[0]
    out_ref[0, 3:CH, :] = jnp.swapaxes(x, 0, 1)


def _assemble(stag, xyzc):
    stag3 = stag.reshape(B, S * K, C)
    xyz3 = xyzc.reshape(B, 3, S * K)
    return pl.pallas_call(
        _asm_body,
        grid=(B, (S * K) // TSK),
        in_specs=[
            pl.BlockSpec((1, TSK, C), lambda b, t: (b, t, 0)),
            pl.BlockSpec((1, 3, TSK), lambda b, t: (b, 0, t)),
        ],
        out_specs=pl.BlockSpec((1, CH, TSK), lambda b, t: (b, 0, t)),
        out_shape=jax.ShapeDtypeStruct((B, CH, S * K), jnp.float32),
    )(stag3, xyz3)


@jax.jit
def _pipeline(xyz, new_xyz, features):
    xyzT = jnp.transpose(xyz, (0, 2, 1))
    words = _pack_words(new_xyz, xyzT, jnp.asarray(_SEG))
    featT = jnp.transpose(features, (0, 2, 1)).reshape(B * N, C)
    xyzp = jnp.concatenate(
        [xyz, jnp.zeros((B, N, 1), jnp.float32)], axis=-1).reshape(-1)
    nxp = jnp.pad(new_xyz.reshape(ROWS, 3) * (1.0 / RADIUS),
                  ((0, 0), (0, 13))).reshape(-1)
    merged = _sc_group()(words, nxp, xyzp, featT)
    stagf = merged[:ROWS * K].reshape(B, S, K, C)
    featc = jnp.transpose(stagf, (0, 3, 1, 2))
    xyzc = merged[ROWS * K:ROWS * K + XROWS].reshape(B, 3, S, K)
    return jnp.concatenate([xyzc, featc], axis=1)


def kernel(xyz, new_xyz, features):
    return _pipeline(xyz, new_xyz, features)
